# baseline (speedup 1.0000x reference)
import functools

import numpy as np
import jax
import jax.numpy as jnp
from jax import lax
from jax.experimental import pallas as pl
from jax.experimental.pallas import tpu as pltpu
from jax.experimental.pallas import tpu_sc as plsc

N = 10000
E = 320000
D_IN = 128
DH = 64
L = 8
ALPHA = 0.1
THETA = 0.5

NPAD = 10240
NC, NS = 2, 16
NW = NC * NS
B = 128
NB = 80
G = 4
NG = NB // G
EPAD = NW * NB * B
EPT = NB * B
RPT = NPAD // NS
RB = 1024
DEGW = 16


@functools.cache
def _sc_kernels():
    mesh = plsc.VectorSubcoreMesh(core_axis_name="c", subcore_axis_name="s")
    params = pltpu.CompilerParams(use_tc_tiling_on_sc=False)

    @functools.partial(
        pl.kernel,
        out_type=jax.ShapeDtypeStruct((NC * NPAD, DEGW), jnp.float32),
        mesh=mesh,
        compiler_params=params,
        scratch_types=[
            pltpu.VMEM((NB, B), jnp.int32),
            pltpu.VMEM((B, DEGW), jnp.float32),
            pltpu.VMEM_SHARED((NPAD, DEGW), jnp.float32),
            pltpu.SemaphoreType.DMA,
        ],
    )
    def deg_kernel(dstR, ones2, deg_out, dst_v, ones_v, deg_sh, sem_s):
        cid = lax.axis_index("c")
        sid = lax.axis_index("s")
        wid = sid * NC + cid
        pltpu.sync_copy(dstR.at[pl.ds(wid * NB, NB)], dst_v)
        pltpu.sync_copy(ones2.at[pl.ds(0, B)], ones_v)
        pltpu.sync_copy(ones2, deg_sh.at[pl.ds(sid * RPT, RPT)])
        plsc.subcore_barrier()

        WDEG = 8

        def body(j, _):
            pltpu.async_copy(ones_v, deg_sh.at[dst_v.at[j]], sem_s, add=True)

            @pl.when(j >= WDEG)
            def _():
                pltpu.make_async_copy(ones_v, deg_sh.at[dst_v.at[j]], sem_s).wait()
            return ()

        lax.fori_loop(0, NB, body, ())

        def drain(j, _):
            pltpu.make_async_copy(ones_v, deg_sh.at[dst_v.at[0]], sem_s).wait()
            return ()

        lax.fori_loop(0, WDEG, drain, ())
        plsc.subcore_barrier()
        pltpu.sync_copy(
            deg_sh.at[pl.ds(sid * RPT, RPT)],
            deg_out.at[pl.ds(cid * NPAD + sid * RPT, RPT)],
        )

    @functools.partial(
        pl.kernel,
        out_type=jax.ShapeDtypeStruct((NC * NPAD, DH), jnp.float32),
        mesh=mesh,
        compiler_params=params,
        scratch_types=[
            pltpu.VMEM((NB, B), jnp.int32),
            pltpu.VMEM((NB, B), jnp.int32),
            pltpu.VMEM((G, B, DH), jnp.float32),
            pltpu.VMEM((G, B, DH), jnp.float32),
            pltpu.VMEM_SHARED((NPAD, DH), jnp.float32),
            pltpu.SemaphoreType.DMA,
            pltpu.SemaphoreType.DMA,
            pltpu.SemaphoreType.DMA,
            pltpu.SemaphoreType.DMA,
        ],
    )
    def prop_kernel(hs, srcR, dstR, s_out, src_v, dst_v, bank_a, bank_b,
                    agg_sh, gsem_a, gsem_b, ssem_a, ssem_b):
        cid = lax.axis_index("c")
        sid = lax.axis_index("s")
        wid = sid * NC + cid
        pltpu.sync_copy(srcR.at[pl.ds(wid * NB, NB)], src_v)
        pltpu.sync_copy(dstR.at[pl.ds(wid * NB, NB)], dst_v)
        pltpu.sync_copy(hs.at[pl.ds(sid * RPT, RPT)],
                        agg_sh.at[pl.ds(sid * RPT, RPT)])
        plsc.subcore_barrier()

        def fire_gathers(g, bank, sem):
            for q in range(G):
                pltpu.async_copy(hs.at[src_v.at[g * G + q]], bank.at[q], sem)

        def drain_gathers(g, bank, sem):
            for q in range(G):
                pltpu.make_async_copy(hs.at[src_v.at[g * G + q]], bank.at[q],
                                      sem).wait()

        def fire_scatters(g, bank, sem):
            for q in range(G):
                pltpu.async_copy(bank.at[q], agg_sh.at[dst_v.at[g * G + q]],
                                 sem, add=True)

        def drain_scatters(g, bank, sem):
            for q in range(G):
                pltpu.make_async_copy(bank.at[q], agg_sh.at[dst_v.at[g * G + q]],
                                      sem).wait()

        fire_gathers(0, bank_a, gsem_a)

        def body(g, _):
            even = g % 2 == 0

            @pl.when(even)
            def _():
                drain_gathers(g, bank_a, gsem_a)
                fire_scatters(g, bank_a, ssem_a)

                @pl.when(g + 1 < NG)
                def _():
                    @pl.when(g > 0)
                    def _():
                        drain_scatters(g - 1, bank_b, ssem_b)
                    fire_gathers(g + 1, bank_b, gsem_b)

            @pl.when(jnp.logical_not(even))
            def _():
                drain_gathers(g, bank_b, gsem_b)
                fire_scatters(g, bank_b, ssem_b)

                @pl.when(g + 1 < NG)
                def _():
                    drain_scatters(g - 1, bank_a, ssem_a)
                    fire_gathers(g + 1, bank_a, gsem_a)

            return ()

        lax.fori_loop(0, NG, body, ())
        drain_scatters(NG - 2, bank_a, ssem_a)
        drain_scatters(NG - 1, bank_b, ssem_b)
        plsc.subcore_barrier()
        pltpu.sync_copy(
            agg_sh.at[pl.ds(sid * RPT, RPT)],
            s_out.at[pl.ds(cid * NPAD + sid * RPT, RPT)],
        )

    return deg_kernel, prop_kernel


def _pre_body(xb, w0b, b0b, degb, x0b, hsb, Db):
    h = jnp.dot(xb[...], w0b[...], preferred_element_type=jnp.float32)
    h = jnp.maximum(h + b0b[0:1, :], 0.0)
    deg = degb[0, :, :1] + degb[1, :, :1] - 1.0
    D = jnp.broadcast_to(lax.rsqrt(deg), (RB, DH))
    x0b[...] = h
    Db[...] = D
    hsb[...] = D * h


@functools.cache
def _pre_kernel():
    return pl.pallas_call(
        _pre_body,
        grid=(NPAD // RB,),
        in_specs=[
            pl.BlockSpec((RB, D_IN), lambda i: (i, 0)),
            pl.BlockSpec((D_IN, DH), lambda i: (0, 0)),
            pl.BlockSpec((8, DH), lambda i: (0, 0)),
            pl.BlockSpec((2, RB, DEGW), lambda i: (0, i, 0)),
        ],
        out_specs=[
            pl.BlockSpec((RB, DH), lambda i: (i, 0)),
            pl.BlockSpec((RB, DH), lambda i: (i, 0)),
            pl.BlockSpec((RB, DH), lambda i: (i, 0)),
        ],
        out_shape=[jax.ShapeDtypeStruct((NPAD, DH), jnp.float32)] * 3,
    )


def _upd_body(beta, last, sb, hsb, x0b, Db, wcb, w1b, b1b, ob):
    S = sb[0] + sb[1] - hsb[...]
    t = (1.0 - ALPHA) * (Db[...] * S) + ALPHA * x0b[...]
    u = (1.0 - beta) * t + beta * jnp.dot(
        t, wcb[...], preferred_element_type=jnp.float32)
    h = jnp.maximum(u, 0.0)
    if last:
        ob[...] = jnp.dot(h, w1b[...],
                          preferred_element_type=jnp.float32) + b1b[0:1, :]
    else:
        ob[...] = Db[...] * h


@functools.cache
def _upd_kernel(beta, last):
    return pl.pallas_call(
        functools.partial(_upd_body, beta, last),
        grid=(NPAD // RB,),
        in_specs=[
            pl.BlockSpec((2, RB, DH), lambda i: (0, i, 0)),
            pl.BlockSpec((RB, DH), lambda i: (i, 0)),
            pl.BlockSpec((RB, DH), lambda i: (i, 0)),
            pl.BlockSpec((RB, DH), lambda i: (i, 0)),
            pl.BlockSpec((DH, DH), lambda i: (0, 0)),
            pl.BlockSpec((DH, DH), lambda i: (0, 0)),
            pl.BlockSpec((8, DH), lambda i: (0, 0)),
        ],
        out_specs=pl.BlockSpec((RB, DH), lambda i: (i, 0)),
        out_shape=jax.ShapeDtypeStruct((NPAD, DH), jnp.float32),
    )


def kernel(x, edge_index, W0, b0, W1, b1, Wc):
    deg_kernel, prop_kernel = _sc_kernels()

    src = edge_index[0].astype(jnp.int32)
    dst = edge_index[1].astype(jnp.int32)
    srcR = jnp.concatenate(
        [src, jnp.zeros((EPAD - E,), jnp.int32)]).reshape(EPAD // B, B)
    dstR = jnp.concatenate(
        [dst, jnp.full((EPAD - E,), NPAD - 1, jnp.int32)]).reshape(EPAD // B, B)
    xp = jnp.pad(x, ((0, NPAD - N), (0, 0)))
    ones2 = jnp.ones((RPT, DEGW), jnp.float32)
    b0_8 = jnp.tile(b0[None, :], (8, 1))
    b1_8 = jnp.tile(b1[None, :], (8, 1))

    deg = deg_kernel(dstR, ones2).reshape(NC, NPAD, DEGW)
    x0, hs, D = _pre_kernel()(xp, W0, b0_8, deg)
    for l in range(L):
        beta = float(np.log(THETA / (l + 1) + 1.0))
        s = prop_kernel(hs, srcR, dstR).reshape(NC, NPAD, DH)
        hs = _upd_kernel(beta, l == L - 1)(s, hs, x0, D, Wc[l], W1, b1_8)
    return hs[:N]


# speedup vs baseline: 2.4277x; 2.4277x over previous
import functools

import numpy as np
import jax
import jax.numpy as jnp
from jax import lax
from jax.experimental import pallas as pl
from jax.experimental.pallas import tpu as pltpu
from jax.experimental.pallas import tpu_sc as plsc

N = 10000
E = 320000
D_IN = 128
DH = 64
DHH = DH // 2
L = 8
ALPHA = 0.1
THETA = 0.5

NPAD = 10240
NC, NS = 2, 16
NW = NC * NS
B = 128
NBD = 80
EPAD = NW * NBD * B
NB = EPAD // (NS * B)
G = 4
NG = NB // G
RPT = NPAD // NS
RB = 1024
DEGW = 16


@functools.cache
def _sc_kernels():
    mesh = plsc.VectorSubcoreMesh(core_axis_name="c", subcore_axis_name="s")
    params = pltpu.CompilerParams(use_tc_tiling_on_sc=False)

    @functools.partial(
        pl.kernel,
        out_type=jax.ShapeDtypeStruct((NC * NPAD, DEGW), jnp.float32),
        mesh=mesh,
        compiler_params=params,
        scratch_types=[
            pltpu.VMEM((NBD, B), jnp.int32),
            pltpu.VMEM((B, DEGW), jnp.float32),
            pltpu.VMEM_SHARED((NPAD, DEGW), jnp.float32),
            pltpu.SemaphoreType.DMA,
        ],
    )
    def deg_kernel(dstR, ones2, deg_out, dst_v, ones_v, deg_sh, sem_s):
        cid = lax.axis_index("c")
        sid = lax.axis_index("s")
        wid = sid * NC + cid
        pltpu.sync_copy(dstR.at[pl.ds(wid * NBD, NBD)], dst_v)
        pltpu.sync_copy(ones2.at[pl.ds(0, B)], ones_v)
        pltpu.sync_copy(ones2, deg_sh.at[pl.ds(sid * RPT, RPT)])
        plsc.subcore_barrier()

        WDEG = 8

        def body(j, _):
            pltpu.async_copy(ones_v, deg_sh.at[dst_v.at[j]], sem_s, add=True)

            @pl.when(j >= WDEG)
            def _():
                pltpu.make_async_copy(ones_v, deg_sh.at[dst_v.at[j]], sem_s).wait()
            return ()

        lax.fori_loop(0, NBD, body, ())

        def drain(j, _):
            pltpu.make_async_copy(ones_v, deg_sh.at[dst_v.at[0]], sem_s).wait()
            return ()

        lax.fori_loop(0, WDEG, drain, ())
        plsc.subcore_barrier()
        pltpu.sync_copy(
            deg_sh.at[pl.ds(sid * RPT, RPT)],
            deg_out.at[pl.ds(cid * NPAD + sid * RPT, RPT)],
        )

    @functools.partial(
        pl.kernel,
        out_type=jax.ShapeDtypeStruct((NC, NPAD, DHH), jnp.float32),
        mesh=mesh,
        compiler_params=params,
        scratch_types=[
            pltpu.VMEM((NB, B), jnp.int32),
            pltpu.VMEM((NB, B), jnp.int32),
            pltpu.VMEM((G, B, DHH), jnp.float32),
            pltpu.VMEM((G, B, DHH), jnp.float32),
            pltpu.VMEM_SHARED((NPAD, DHH), jnp.float32),
            pltpu.VMEM_SHARED((NPAD, DHH), jnp.float32),
            pltpu.SemaphoreType.DMA,
            pltpu.SemaphoreType.DMA,
            pltpu.SemaphoreType.DMA,
            pltpu.SemaphoreType.DMA,
        ],
    )
    def prop_kernel(hs2, srcR, dstR, s_out, src_v, dst_v, bank_a, bank_b,
                    agg_sh, tab, gsem_a, gsem_b, ssem_a, ssem_b):
        cid = lax.axis_index("c")
        sid = lax.axis_index("s")
        pltpu.sync_copy(srcR.at[pl.ds(sid * NB, NB)], src_v)
        pltpu.sync_copy(dstR.at[pl.ds(sid * NB, NB)], dst_v)
        pltpu.sync_copy(hs2.at[cid, pl.ds(sid * RPT, RPT)],
                        tab.at[pl.ds(sid * RPT, RPT)])
        pltpu.sync_copy(hs2.at[cid, pl.ds(sid * RPT, RPT)],
                        agg_sh.at[pl.ds(sid * RPT, RPT)])
        plsc.subcore_barrier()

        def fire_gathers(g, bank, sem):
            for q in range(G):
                pltpu.async_copy(tab.at[src_v.at[g * G + q]], bank.at[q], sem)

        def drain_gathers(g, bank, sem):
            for q in range(G):
                pltpu.make_async_copy(tab.at[src_v.at[g * G + q]], bank.at[q],
                                      sem).wait()

        def fire_scatters(g, bank, sem):
            for q in range(G):
                pltpu.async_copy(bank.at[q], agg_sh.at[dst_v.at[g * G + q]],
                                 sem, add=True)

        def drain_scatters(g, bank, sem):
            for q in range(G):
                pltpu.make_async_copy(bank.at[q], agg_sh.at[dst_v.at[g * G + q]],
                                      sem).wait()

        fire_gathers(0, bank_a, gsem_a)

        def body(g, _):
            even = g % 2 == 0

            @pl.when(even)
            def _():
                drain_gathers(g, bank_a, gsem_a)
                fire_scatters(g, bank_a, ssem_a)

                @pl.when(g + 1 < NG)
                def _():
                    @pl.when(g > 0)
                    def _():
                        drain_scatters(g - 1, bank_b, ssem_b)
                    fire_gathers(g + 1, bank_b, gsem_b)

            @pl.when(jnp.logical_not(even))
            def _():
                drain_gathers(g, bank_b, gsem_b)
                fire_scatters(g, bank_b, ssem_b)

                @pl.when(g + 1 < NG)
                def _():
                    drain_scatters(g - 1, bank_a, ssem_a)
                    fire_gathers(g + 1, bank_a, gsem_a)

            return ()

        lax.fori_loop(0, NG, body, ())
        drain_scatters(NG - 2, bank_a, ssem_a)
        drain_scatters(NG - 1, bank_b, ssem_b)
        plsc.subcore_barrier()
        pltpu.sync_copy(
            agg_sh.at[pl.ds(sid * RPT, RPT)],
            s_out.at[cid, pl.ds(sid * RPT, RPT)],
        )

    return deg_kernel, prop_kernel


def _pre_body(xb, w0b, b0b, degb, x0b, hsb, Db):
    h = jnp.dot(xb[...], w0b[...], preferred_element_type=jnp.float32)
    h = jnp.maximum(h + b0b[0:1, :], 0.0)
    deg = degb[0, :, :1] + degb[1, :, :1] - 1.0
    D = jnp.broadcast_to(lax.rsqrt(deg), (RB, DH))
    x0b[...] = h
    Db[...] = D
    hs = D * h
    hsb[...] = jnp.stack([hs[:, :DHH], hs[:, DHH:]], axis=0)


@functools.cache
def _pre_kernel():
    return pl.pallas_call(
        _pre_body,
        grid=(NPAD // RB,),
        in_specs=[
            pl.BlockSpec((RB, D_IN), lambda i: (i, 0)),
            pl.BlockSpec((D_IN, DH), lambda i: (0, 0)),
            pl.BlockSpec((8, DH), lambda i: (0, 0)),
            pl.BlockSpec((2, RB, DEGW), lambda i: (0, i, 0)),
        ],
        out_specs=[
            pl.BlockSpec((RB, DH), lambda i: (i, 0)),
            pl.BlockSpec((2, RB, DHH), lambda i: (0, i, 0)),
            pl.BlockSpec((RB, DH), lambda i: (i, 0)),
        ],
        out_shape=[
            jax.ShapeDtypeStruct((NPAD, DH), jnp.float32),
            jax.ShapeDtypeStruct((NC, NPAD, DHH), jnp.float32),
            jax.ShapeDtypeStruct((NPAD, DH), jnp.float32),
        ],
    )


def _upd_body(beta, last, sb, x0b, Db, wcb, w1b, b1b, ob):
    S = jnp.concatenate([sb[0], sb[1]], axis=1)
    t = (1.0 - ALPHA) * (Db[...] * S) + ALPHA * x0b[...]
    u = (1.0 - beta) * t + beta * jnp.dot(
        t, wcb[...], preferred_element_type=jnp.float32)
    h = jnp.maximum(u, 0.0)
    if last:
        ob[...] = jnp.dot(h, w1b[...],
                          preferred_element_type=jnp.float32) + b1b[0:1, :]
    else:
        hs = Db[...] * h
        ob[...] = jnp.stack([hs[:, :DHH], hs[:, DHH:]], axis=0)


@functools.cache
def _upd_kernel(beta, last):
    if last:
        out_spec = pl.BlockSpec((RB, DH), lambda i: (i, 0))
        out_shape = jax.ShapeDtypeStruct((NPAD, DH), jnp.float32)
    else:
        out_spec = pl.BlockSpec((2, RB, DHH), lambda i: (0, i, 0))
        out_shape = jax.ShapeDtypeStruct((NC, NPAD, DHH), jnp.float32)
    return pl.pallas_call(
        functools.partial(_upd_body, beta, last),
        grid=(NPAD // RB,),
        in_specs=[
            pl.BlockSpec((2, RB, DHH), lambda i: (0, i, 0)),
            pl.BlockSpec((RB, DH), lambda i: (i, 0)),
            pl.BlockSpec((RB, DH), lambda i: (i, 0)),
            pl.BlockSpec((DH, DH), lambda i: (0, 0)),
            pl.BlockSpec((DH, DH), lambda i: (0, 0)),
            pl.BlockSpec((8, DH), lambda i: (0, 0)),
        ],
        out_specs=out_spec,
        out_shape=out_shape,
    )


def kernel(x, edge_index, W0, b0, W1, b1, Wc):
    deg_kernel, prop_kernel = _sc_kernels()

    src = edge_index[0].astype(jnp.int32)
    dst = edge_index[1].astype(jnp.int32)
    srcR = jnp.concatenate(
        [src, jnp.zeros((EPAD - E,), jnp.int32)]).reshape(EPAD // B, B)
    dstR = jnp.concatenate(
        [dst, jnp.full((EPAD - E,), NPAD - 1, jnp.int32)]).reshape(EPAD // B, B)
    xp = jnp.pad(x, ((0, NPAD - N), (0, 0)))
    ones2 = jnp.ones((RPT, DEGW), jnp.float32)
    b0_8 = jnp.tile(b0[None, :], (8, 1))
    b1_8 = jnp.tile(b1[None, :], (8, 1))

    deg = deg_kernel(dstR, ones2).reshape(NC, NPAD, DEGW)
    x0, hs2, D = _pre_kernel()(xp, W0, b0_8, deg)
    for l in range(L):
        beta = float(np.log(THETA / (l + 1) + 1.0))
        s2 = prop_kernel(hs2, srcR, dstR)
        hs2 = _upd_kernel(beta, l == L - 1)(s2, x0, D, Wc[l], W1, b1_8)
    return hs2[:N]


# speedup vs baseline: 2.4666x; 1.0160x over previous
import functools

import numpy as np
import jax
import jax.numpy as jnp
from jax import lax
from jax.experimental import pallas as pl
from jax.experimental.pallas import tpu as pltpu
from jax.experimental.pallas import tpu_sc as plsc

N = 10000
E = 320000
D_IN = 128
DH = 64
DHH = DH // 2
L = 8
ALPHA = 0.1
THETA = 0.5

NPAD = 10240
NC, NS = 2, 16
NW = NC * NS
B = 128
NBD = 80
EPAD = NW * NBD * B
NB = EPAD // (NS * B)
G = 4
NG = NB // G
RPT = NPAD // NS
RB = 2048
DEGW = 16


@functools.cache
def _sc_kernels():
    mesh = plsc.VectorSubcoreMesh(core_axis_name="c", subcore_axis_name="s")
    params = pltpu.CompilerParams(use_tc_tiling_on_sc=False)

    @functools.partial(
        pl.kernel,
        out_type=jax.ShapeDtypeStruct((NC, NPAD, DEGW), jnp.float32),
        mesh=mesh,
        compiler_params=params,
        scratch_types=[
            pltpu.VMEM((NBD, B), jnp.int32),
            pltpu.VMEM((B, DEGW), jnp.float32),
            pltpu.VMEM_SHARED((NPAD, DEGW), jnp.float32),
            pltpu.SemaphoreType.DMA,
        ],
    )
    def deg_kernel(dstR, ones2, deg_out, dst_v, ones_v, deg_sh, sem_s):
        cid = lax.axis_index("c")
        sid = lax.axis_index("s")
        wid = sid * NC + cid
        pltpu.sync_copy(dstR.at[pl.ds(wid * NBD, NBD)], dst_v)
        pltpu.sync_copy(ones2.at[pl.ds(0, B)], ones_v)
        pltpu.sync_copy(ones2, deg_sh.at[pl.ds(sid * RPT, RPT)])
        plsc.subcore_barrier()

        WDEG = 8

        def body(j, _):
            pltpu.async_copy(ones_v, deg_sh.at[dst_v.at[j]], sem_s, add=True)

            @pl.when(j >= WDEG)
            def _():
                pltpu.make_async_copy(ones_v, deg_sh.at[dst_v.at[j]], sem_s).wait()
            return ()

        lax.fori_loop(0, NBD, body, ())

        def drain(j, _):
            pltpu.make_async_copy(ones_v, deg_sh.at[dst_v.at[0]], sem_s).wait()
            return ()

        lax.fori_loop(0, WDEG, drain, ())
        plsc.subcore_barrier()
        pltpu.sync_copy(
            deg_sh.at[pl.ds(sid * RPT, RPT)],
            deg_out.at[cid, pl.ds(sid * RPT, RPT)],
        )

    @functools.partial(
        pl.kernel,
        out_type=jax.ShapeDtypeStruct((NC, NPAD, DHH), jnp.float32),
        mesh=mesh,
        compiler_params=params,
        scratch_types=[
            pltpu.VMEM((NB, B), jnp.int32),
            pltpu.VMEM((NB, B), jnp.int32),
            pltpu.VMEM((G, B, DHH), jnp.float32),
            pltpu.VMEM((G, B, DHH), jnp.float32),
            pltpu.VMEM_SHARED((NPAD, DHH), jnp.float32),
            pltpu.VMEM_SHARED((NPAD, DHH), jnp.float32),
            pltpu.SemaphoreType.DMA,
            pltpu.SemaphoreType.DMA,
            pltpu.SemaphoreType.DMA,
            pltpu.SemaphoreType.DMA,
        ],
    )
    def prop_kernel(hs2, srcR, dstR, s_out, src_v, dst_v, bank_a, bank_b,
                    agg_sh, tab, gsem_a, gsem_b, ssem_a, ssem_b):
        cid = lax.axis_index("c")
        sid = lax.axis_index("s")
        pltpu.sync_copy(srcR.at[pl.ds(sid * NB, NB)], src_v)
        pltpu.sync_copy(dstR.at[pl.ds(sid * NB, NB)], dst_v)
        pltpu.sync_copy(hs2.at[cid, pl.ds(sid * RPT, RPT)],
                        tab.at[pl.ds(sid * RPT, RPT)])
        pltpu.sync_copy(hs2.at[cid, pl.ds(sid * RPT, RPT)],
                        agg_sh.at[pl.ds(sid * RPT, RPT)])
        plsc.subcore_barrier()

        def fire_gathers(g, bank, sem):
            for q in range(G):
                pltpu.async_copy(tab.at[src_v.at[g * G + q]], bank.at[q], sem)

        def drain_gathers(g, bank, sem):
            for q in range(G):
                pltpu.make_async_copy(tab.at[src_v.at[g * G + q]], bank.at[q],
                                      sem).wait()

        def fire_scatters(g, bank, sem):
            for q in range(G):
                pltpu.async_copy(bank.at[q], agg_sh.at[dst_v.at[g * G + q]],
                                 sem, add=True)

        def drain_scatters(g, bank, sem):
            for q in range(G):
                pltpu.make_async_copy(bank.at[q], agg_sh.at[dst_v.at[g * G + q]],
                                      sem).wait()

        fire_gathers(0, bank_a, gsem_a)

        def body(g, _):
            even = g % 2 == 0

            @pl.when(even)
            def _():
                drain_gathers(g, bank_a, gsem_a)
                fire_scatters(g, bank_a, ssem_a)

                @pl.when(g + 1 < NG)
                def _():
                    @pl.when(g > 0)
                    def _():
                        drain_scatters(g - 1, bank_b, ssem_b)
                    fire_gathers(g + 1, bank_b, gsem_b)

            @pl.when(jnp.logical_not(even))
            def _():
                drain_gathers(g, bank_b, gsem_b)
                fire_scatters(g, bank_b, ssem_b)

                @pl.when(g + 1 < NG)
                def _():
                    drain_scatters(g - 1, bank_a, ssem_a)
                    fire_gathers(g + 1, bank_a, gsem_a)

            return ()

        lax.fori_loop(0, NG, body, ())
        drain_scatters(NG - 2, bank_a, ssem_a)
        drain_scatters(NG - 1, bank_b, ssem_b)
        plsc.subcore_barrier()
        pltpu.sync_copy(
            agg_sh.at[pl.ds(sid * RPT, RPT)],
            s_out.at[cid, pl.ds(sid * RPT, RPT)],
        )

    return deg_kernel, prop_kernel


def _pre_body(xb, w0b, b0b, degb, x0b, hsb, Db):
    h = jnp.dot(xb[...], w0b[...], preferred_element_type=jnp.float32)
    h = jnp.maximum(h + b0b[0:1, :], 0.0)
    deg = degb[0, :, :1] + degb[1, :, :1] - 1.0
    D = jnp.broadcast_to(lax.rsqrt(deg), (RB, DH))
    x0b[...] = h
    Db[...] = D
    hs = D * h
    hsb[...] = jnp.stack([hs[:, :DHH], hs[:, DHH:]], axis=0)


@functools.cache
def _pre_kernel():
    return pl.pallas_call(
        _pre_body,
        grid=(NPAD // RB,),
        in_specs=[
            pl.BlockSpec((RB, D_IN), lambda i: (i, 0)),
            pl.BlockSpec((D_IN, DH), lambda i: (0, 0)),
            pl.BlockSpec((8, DH), lambda i: (0, 0)),
            pl.BlockSpec((2, RB, DEGW), lambda i: (0, i, 0)),
        ],
        out_specs=[
            pl.BlockSpec((RB, DH), lambda i: (i, 0)),
            pl.BlockSpec((2, RB, DHH), lambda i: (0, i, 0)),
            pl.BlockSpec((RB, DH), lambda i: (i, 0)),
        ],
        out_shape=[
            jax.ShapeDtypeStruct((NPAD, DH), jnp.float32),
            jax.ShapeDtypeStruct((NC, NPAD, DHH), jnp.float32),
            jax.ShapeDtypeStruct((NPAD, DH), jnp.float32),
        ],
    )


def _upd_body(beta, last, sb, x0b, Db, wcb, w1b, b1b, ob):
    S = jnp.concatenate([sb[0], sb[1]], axis=1)
    t = (1.0 - ALPHA) * (Db[...] * S) + ALPHA * x0b[...]
    u = (1.0 - beta) * t + beta * jnp.dot(
        t, wcb[...], preferred_element_type=jnp.float32)
    h = jnp.maximum(u, 0.0)
    if last:
        ob[...] = jnp.dot(h, w1b[...],
                          preferred_element_type=jnp.float32) + b1b[0:1, :]
    else:
        hs = Db[...] * h
        ob[...] = jnp.stack([hs[:, :DHH], hs[:, DHH:]], axis=0)


@functools.cache
def _upd_kernel(beta, last):
    if last:
        out_spec = pl.BlockSpec((RB, DH), lambda i: (i, 0))
        out_shape = jax.ShapeDtypeStruct((NPAD, DH), jnp.float32)
    else:
        out_spec = pl.BlockSpec((2, RB, DHH), lambda i: (0, i, 0))
        out_shape = jax.ShapeDtypeStruct((NC, NPAD, DHH), jnp.float32)
    return pl.pallas_call(
        functools.partial(_upd_body, beta, last),
        grid=(NPAD // RB,),
        in_specs=[
            pl.BlockSpec((2, RB, DHH), lambda i: (0, i, 0)),
            pl.BlockSpec((RB, DH), lambda i: (i, 0)),
            pl.BlockSpec((RB, DH), lambda i: (i, 0)),
            pl.BlockSpec((DH, DH), lambda i: (0, 0)),
            pl.BlockSpec((DH, DH), lambda i: (0, 0)),
            pl.BlockSpec((8, DH), lambda i: (0, 0)),
        ],
        out_specs=out_spec,
        out_shape=out_shape,
    )


def kernel(x, edge_index, W0, b0, W1, b1, Wc):
    deg_kernel, prop_kernel = _sc_kernels()

    src = edge_index[0].astype(jnp.int32)
    dst = edge_index[1].astype(jnp.int32)
    srcR = jnp.concatenate(
        [src, jnp.zeros((EPAD - E,), jnp.int32)]).reshape(EPAD // B, B)
    dstR = jnp.concatenate(
        [dst, jnp.full((EPAD - E,), NPAD - 1, jnp.int32)]).reshape(EPAD // B, B)
    xp = jnp.pad(x, ((0, NPAD - N), (0, 0)))
    ones2 = jnp.ones((RPT, DEGW), jnp.float32)
    b0_8 = jnp.tile(b0[None, :], (8, 1))
    b1_8 = jnp.tile(b1[None, :], (8, 1))

    deg = deg_kernel(dstR, ones2)
    x0, hs2, D = _pre_kernel()(xp, W0, b0_8, deg)
    for l in range(L):
        beta = float(np.log(THETA / (l + 1) + 1.0))
        s2 = prop_kernel(hs2, srcR, dstR)
        hs2 = _upd_kernel(beta, l == L - 1)(s2, x0, D, Wc[l], W1, b1_8)
    return hs2[:N]


# speedup vs baseline: 2.5215x; 1.0223x over previous
import functools

import numpy as np
import jax
import jax.numpy as jnp
from jax import lax
from jax.experimental import pallas as pl
from jax.experimental.pallas import tpu as pltpu
from jax.experimental.pallas import tpu_sc as plsc

N = 10000
E = 320000
D_IN = 128
DH = 64
DHH = DH // 2
L = 8
ALPHA = 0.1
THETA = 0.5

NPAD = 10240
NC, NS = 2, 16
NW = NC * NS
B = 128
NBD = 80
EPAD = NW * NBD * B
NB = EPAD // (NS * B)
G = 4
NG = NB // G
RPT = NPAD // NS
RB = 2048
DEGW = 16


@functools.cache
def _sc_kernels():
    mesh = plsc.VectorSubcoreMesh(core_axis_name="c", subcore_axis_name="s")
    params = pltpu.CompilerParams(use_tc_tiling_on_sc=False)

    @functools.partial(
        pl.kernel,
        out_type=jax.ShapeDtypeStruct((NC, NPAD, DEGW), jnp.float32),
        mesh=mesh,
        compiler_params=params,
        scratch_types=[
            pltpu.VMEM((NBD, B), jnp.int32),
            pltpu.VMEM((B, DEGW), jnp.float32),
            pltpu.VMEM_SHARED((NPAD, DEGW), jnp.float32),
            pltpu.SemaphoreType.DMA,
        ],
    )
    def deg_kernel(dstR, ones2, deg_out, dst_v, ones_v, deg_sh, sem_s):
        cid = lax.axis_index("c")
        sid = lax.axis_index("s")
        wid = sid * NC + cid
        pltpu.sync_copy(dstR.at[pl.ds(wid * NBD, NBD)], dst_v)
        pltpu.sync_copy(ones2.at[pl.ds(0, B)], ones_v)
        pltpu.sync_copy(ones2, deg_sh.at[pl.ds(sid * RPT, RPT)])
        plsc.subcore_barrier()

        WDEG = 8

        def body(j, _):
            pltpu.async_copy(ones_v, deg_sh.at[dst_v.at[j]], sem_s, add=True)

            @pl.when(j >= WDEG)
            def _():
                pltpu.make_async_copy(ones_v, deg_sh.at[dst_v.at[j]], sem_s).wait()
            return ()

        lax.fori_loop(0, NBD, body, ())

        def drain(j, _):
            pltpu.make_async_copy(ones_v, deg_sh.at[dst_v.at[0]], sem_s).wait()
            return ()

        lax.fori_loop(0, WDEG, drain, ())
        plsc.subcore_barrier()
        pltpu.sync_copy(
            deg_sh.at[pl.ds(sid * RPT, RPT)],
            deg_out.at[cid, pl.ds(sid * RPT, RPT)],
        )

    @functools.partial(
        pl.kernel,
        out_type=jax.ShapeDtypeStruct((NC, NPAD, DHH), jnp.float32),
        mesh=mesh,
        compiler_params=params,
        scratch_types=[
            pltpu.VMEM((NB, B), jnp.int32),
            pltpu.VMEM((NB, B), jnp.int32),
            pltpu.VMEM((G, B, DHH), jnp.float32),
            pltpu.VMEM((G, B, DHH), jnp.float32),
            pltpu.VMEM_SHARED((NPAD, DHH), jnp.float32),
            pltpu.VMEM_SHARED((NPAD, DHH), jnp.float32),
            pltpu.SemaphoreType.DMA,
            pltpu.SemaphoreType.DMA,
            pltpu.SemaphoreType.DMA,
            pltpu.SemaphoreType.DMA,
        ],
    )
    def prop_kernel(hs2, srcR, dstR, s_out, src_v, dst_v, bank_a, bank_b,
                    agg_sh, tab, gsem_a, gsem_b, ssem_a, ssem_b):
        cid = lax.axis_index("c")
        sid = lax.axis_index("s")
        c1 = pltpu.async_copy(srcR.at[pl.ds(sid * NB, NB)], src_v, gsem_a)
        c2 = pltpu.async_copy(dstR.at[pl.ds(sid * NB, NB)], dst_v, gsem_b)
        c3 = pltpu.async_copy(hs2.at[cid, pl.ds(sid * RPT, RPT)],
                              tab.at[pl.ds(sid * RPT, RPT)], ssem_a)
        c4 = pltpu.async_copy(hs2.at[cid, pl.ds(sid * RPT, RPT)],
                              agg_sh.at[pl.ds(sid * RPT, RPT)], ssem_b)
        c1.wait()
        c2.wait()
        c3.wait()
        c4.wait()
        plsc.subcore_barrier()

        def fire_gathers(g, bank, sem):
            for q in range(G):
                pltpu.async_copy(tab.at[src_v.at[g * G + q]], bank.at[q], sem)

        def drain_gathers(g, bank, sem):
            for q in range(G):
                pltpu.make_async_copy(tab.at[src_v.at[g * G + q]], bank.at[q],
                                      sem).wait()

        def fire_scatters(g, bank, sem):
            for q in range(G):
                pltpu.async_copy(bank.at[q], agg_sh.at[dst_v.at[g * G + q]],
                                 sem, add=True)

        def drain_scatters(g, bank, sem):
            for q in range(G):
                pltpu.make_async_copy(bank.at[q], agg_sh.at[dst_v.at[g * G + q]],
                                      sem).wait()

        fire_gathers(0, bank_a, gsem_a)

        def body(g, _):
            even = g % 2 == 0

            @pl.when(even)
            def _():
                drain_gathers(g, bank_a, gsem_a)
                fire_scatters(g, bank_a, ssem_a)

                @pl.when(g + 1 < NG)
                def _():
                    @pl.when(g > 0)
                    def _():
                        drain_scatters(g - 1, bank_b, ssem_b)
                    fire_gathers(g + 1, bank_b, gsem_b)

            @pl.when(jnp.logical_not(even))
            def _():
                drain_gathers(g, bank_b, gsem_b)
                fire_scatters(g, bank_b, ssem_b)

                @pl.when(g + 1 < NG)
                def _():
                    drain_scatters(g - 1, bank_a, ssem_a)
                    fire_gathers(g + 1, bank_a, gsem_a)

            return ()

        lax.fori_loop(0, NG, body, ())
        drain_scatters(NG - 2, bank_a, ssem_a)
        drain_scatters(NG - 1, bank_b, ssem_b)
        plsc.subcore_barrier()
        pltpu.sync_copy(
            agg_sh.at[pl.ds(sid * RPT, RPT)],
            s_out.at[cid, pl.ds(sid * RPT, RPT)],
        )

    return deg_kernel, prop_kernel


def _pre_body(xb, w0b, b0b, degb, x0b, hsb, Db):
    h = jnp.dot(xb[...], w0b[...], preferred_element_type=jnp.float32)
    h = jnp.maximum(h + b0b[0:1, :], 0.0)
    deg = degb[0, :, :1] + degb[1, :, :1] - 1.0
    D = jnp.broadcast_to(lax.rsqrt(deg), (RB, DH))
    x0b[...] = h
    Db[...] = D
    hs = D * h
    hsb[...] = jnp.stack([hs[:, :DHH], hs[:, DHH:]], axis=0)


@functools.cache
def _pre_kernel():
    return pl.pallas_call(
        _pre_body,
        grid=(NPAD // RB,),
        in_specs=[
            pl.BlockSpec((RB, D_IN), lambda i: (i, 0)),
            pl.BlockSpec((D_IN, DH), lambda i: (0, 0)),
            pl.BlockSpec((8, DH), lambda i: (0, 0)),
            pl.BlockSpec((2, RB, DEGW), lambda i: (0, i, 0)),
        ],
        out_specs=[
            pl.BlockSpec((RB, DH), lambda i: (i, 0)),
            pl.BlockSpec((2, RB, DHH), lambda i: (0, i, 0)),
            pl.BlockSpec((RB, DH), lambda i: (i, 0)),
        ],
        out_shape=[
            jax.ShapeDtypeStruct((NPAD, DH), jnp.float32),
            jax.ShapeDtypeStruct((NC, NPAD, DHH), jnp.float32),
            jax.ShapeDtypeStruct((NPAD, DH), jnp.float32),
        ],
    )


def _upd_body(beta, last, sb, x0b, Db, wcb, w1b, b1b, ob):
    S = jnp.concatenate([sb[0], sb[1]], axis=1)
    t = (1.0 - ALPHA) * (Db[...] * S) + ALPHA * x0b[...]
    u = (1.0 - beta) * t + beta * jnp.dot(
        t, wcb[...], preferred_element_type=jnp.float32)
    h = jnp.maximum(u, 0.0)
    if last:
        ob[...] = jnp.dot(h, w1b[...],
                          preferred_element_type=jnp.float32) + b1b[0:1, :]
    else:
        hs = Db[...] * h
        ob[...] = jnp.stack([hs[:, :DHH], hs[:, DHH:]], axis=0)


@functools.cache
def _upd_kernel(beta, last):
    if last:
        out_spec = pl.BlockSpec((RB, DH), lambda i: (i, 0))
        out_shape = jax.ShapeDtypeStruct((NPAD, DH), jnp.float32)
    else:
        out_spec = pl.BlockSpec((2, RB, DHH), lambda i: (0, i, 0))
        out_shape = jax.ShapeDtypeStruct((NC, NPAD, DHH), jnp.float32)
    return pl.pallas_call(
        functools.partial(_upd_body, beta, last),
        grid=(NPAD // RB,),
        in_specs=[
            pl.BlockSpec((2, RB, DHH), lambda i: (0, i, 0)),
            pl.BlockSpec((RB, DH), lambda i: (i, 0)),
            pl.BlockSpec((RB, DH), lambda i: (i, 0)),
            pl.BlockSpec((DH, DH), lambda i: (0, 0)),
            pl.BlockSpec((DH, DH), lambda i: (0, 0)),
            pl.BlockSpec((8, DH), lambda i: (0, 0)),
        ],
        out_specs=out_spec,
        out_shape=out_shape,
    )


def kernel(x, edge_index, W0, b0, W1, b1, Wc):
    deg_kernel, prop_kernel = _sc_kernels()

    src = edge_index[0].astype(jnp.int32)
    dst = edge_index[1].astype(jnp.int32)
    srcR = jnp.concatenate(
        [src, jnp.zeros((EPAD - E,), jnp.int32)]).reshape(EPAD // B, B)
    dstR = jnp.concatenate(
        [dst, jnp.full((EPAD - E,), NPAD - 1, jnp.int32)]).reshape(EPAD // B, B)
    xp = jnp.pad(x, ((0, NPAD - N), (0, 0)))
    ones2 = jnp.ones((RPT, DEGW), jnp.float32)
    b0_8 = jnp.tile(b0[None, :], (8, 1))
    b1_8 = jnp.tile(b1[None, :], (8, 1))

    deg = deg_kernel(dstR, ones2)
    x0, hs2, D = _pre_kernel()(xp, W0, b0_8, deg)
    for l in range(L):
        beta = float(np.log(THETA / (l + 1) + 1.0))
        s2 = prop_kernel(hs2, srcR, dstR)
        hs2 = _upd_kernel(beta, l == L - 1)(s2, x0, D, Wc[l], W1, b1_8)
    return hs2[:N]


# speedup vs baseline: 2.5637x; 1.0168x over previous
import functools

import numpy as np
import jax
import jax.numpy as jnp
from jax import lax
from jax.experimental import pallas as pl
from jax.experimental.pallas import tpu as pltpu
from jax.experimental.pallas import tpu_sc as plsc

N = 10000
E = 320000
D_IN = 128
DH = 64
DHH = DH // 2
L = 8
ALPHA = 0.1
THETA = 0.5

NPAD = 10240
NC, NS = 2, 16
NW = NC * NS
B = 128
NBD = 80
EPAD = NW * NBD * B
NB = EPAD // (NS * B)
G = 4
NG = NB // G
RPT = NPAD // NS
RB = 2048
DEGW = 16


@functools.cache
def _sc_kernels():
    mesh = plsc.VectorSubcoreMesh(core_axis_name="c", subcore_axis_name="s")
    params = pltpu.CompilerParams(use_tc_tiling_on_sc=False)

    @functools.partial(
        pl.kernel,
        out_type=jax.ShapeDtypeStruct((NC, NPAD, DEGW), jnp.float32),
        mesh=mesh,
        compiler_params=params,
        scratch_types=[
            pltpu.VMEM((NBD, B), jnp.int32),
            pltpu.VMEM((B, DEGW), jnp.float32),
            pltpu.VMEM_SHARED((NPAD, DEGW), jnp.float32),
            pltpu.SemaphoreType.DMA,
        ],
    )
    def deg_kernel(eiR, ones2, deg_out, dst_v, ones_v, deg_sh, sem_s):
        cid = lax.axis_index("c")
        sid = lax.axis_index("s")
        wid = sid * NC + cid
        pltpu.sync_copy(eiR.at[1, pl.ds(wid * NBD, NBD)], dst_v)
        pltpu.sync_copy(ones2.at[pl.ds(0, B)], ones_v)
        pltpu.sync_copy(ones2, deg_sh.at[pl.ds(sid * RPT, RPT)])
        plsc.subcore_barrier()

        WDEG = 8

        def body(j, _):
            pltpu.async_copy(ones_v, deg_sh.at[dst_v.at[j]], sem_s, add=True)

            @pl.when(j >= WDEG)
            def _():
                pltpu.make_async_copy(ones_v, deg_sh.at[dst_v.at[j]], sem_s).wait()
            return ()

        lax.fori_loop(0, NBD, body, ())

        def drain(j, _):
            pltpu.make_async_copy(ones_v, deg_sh.at[dst_v.at[0]], sem_s).wait()
            return ()

        lax.fori_loop(0, WDEG, drain, ())
        plsc.subcore_barrier()
        pltpu.sync_copy(
            deg_sh.at[pl.ds(sid * RPT, RPT)],
            deg_out.at[cid, pl.ds(sid * RPT, RPT)],
        )

    @functools.partial(
        pl.kernel,
        out_type=jax.ShapeDtypeStruct((NC, NPAD, DHH), jnp.float32),
        mesh=mesh,
        compiler_params=params,
        scratch_types=[
            pltpu.VMEM((NB, B), jnp.int32),
            pltpu.VMEM((NB, B), jnp.int32),
            pltpu.VMEM((G, B, DHH), jnp.float32),
            pltpu.VMEM((G, B, DHH), jnp.float32),
            pltpu.VMEM_SHARED((NPAD, DHH), jnp.float32),
            pltpu.VMEM_SHARED((NPAD, DHH), jnp.float32),
            pltpu.SemaphoreType.DMA,
            pltpu.SemaphoreType.DMA,
            pltpu.SemaphoreType.DMA,
            pltpu.SemaphoreType.DMA,
        ],
    )
    def prop_kernel(hs2, eiR, s_out, src_v, dst_v, bank_a, bank_b,
                    agg_sh, tab, gsem_a, gsem_b, ssem_a, ssem_b):
        cid = lax.axis_index("c")
        sid = lax.axis_index("s")
        c1 = pltpu.async_copy(eiR.at[0, pl.ds(sid * NB, NB)], src_v, gsem_a)
        c2 = pltpu.async_copy(eiR.at[1, pl.ds(sid * NB, NB)], dst_v, gsem_b)
        c3 = pltpu.async_copy(hs2.at[cid, pl.ds(sid * RPT, RPT)],
                              tab.at[pl.ds(sid * RPT, RPT)], ssem_a)
        c4 = pltpu.async_copy(hs2.at[cid, pl.ds(sid * RPT, RPT)],
                              agg_sh.at[pl.ds(sid * RPT, RPT)], ssem_b)
        c1.wait()
        c2.wait()
        c3.wait()
        c4.wait()
        plsc.subcore_barrier()

        def fire_gathers(g, bank, sem):
            for q in range(G):
                pltpu.async_copy(tab.at[src_v.at[g * G + q]], bank.at[q], sem)

        def drain_gathers(g, bank, sem):
            for q in range(G):
                pltpu.make_async_copy(tab.at[src_v.at[g * G + q]], bank.at[q],
                                      sem).wait()

        def fire_scatters(g, bank, sem):
            for q in range(G):
                pltpu.async_copy(bank.at[q], agg_sh.at[dst_v.at[g * G + q]],
                                 sem, add=True)

        def drain_scatters(g, bank, sem):
            for q in range(G):
                pltpu.make_async_copy(bank.at[q], agg_sh.at[dst_v.at[g * G + q]],
                                      sem).wait()

        fire_gathers(0, bank_a, gsem_a)

        def body(g, _):
            even = g % 2 == 0

            @pl.when(even)
            def _():
                drain_gathers(g, bank_a, gsem_a)
                fire_scatters(g, bank_a, ssem_a)

                @pl.when(g + 1 < NG)
                def _():
                    @pl.when(g > 0)
                    def _():
                        drain_scatters(g - 1, bank_b, ssem_b)
                    fire_gathers(g + 1, bank_b, gsem_b)

            @pl.when(jnp.logical_not(even))
            def _():
                drain_gathers(g, bank_b, gsem_b)
                fire_scatters(g, bank_b, ssem_b)

                @pl.when(g + 1 < NG)
                def _():
                    drain_scatters(g - 1, bank_a, ssem_a)
                    fire_gathers(g + 1, bank_a, gsem_a)

            return ()

        lax.fori_loop(0, NG, body, ())
        drain_scatters(NG - 2, bank_a, ssem_a)
        drain_scatters(NG - 1, bank_b, ssem_b)
        plsc.subcore_barrier()
        pltpu.sync_copy(
            agg_sh.at[pl.ds(sid * RPT, RPT)],
            s_out.at[cid, pl.ds(sid * RPT, RPT)],
        )

    return deg_kernel, prop_kernel


def _pre_body(xb, w0b, b0b, degb, x0b, hsb, Db):
    h = jnp.dot(xb[...], w0b[...], preferred_element_type=jnp.float32)
    h = jnp.maximum(h + b0b[0:1, :], 0.0)
    deg = degb[0, :, :1] + degb[1, :, :1] - 1.0
    D = jnp.broadcast_to(lax.rsqrt(deg), (RB, DH))
    x0b[...] = h
    Db[...] = D
    hs = D * h
    hsb[...] = jnp.stack([hs[:, :DHH], hs[:, DHH:]], axis=0)


@functools.cache
def _pre_kernel():
    return pl.pallas_call(
        _pre_body,
        grid=(NPAD // RB,),
        in_specs=[
            pl.BlockSpec((RB, D_IN), lambda i: (i, 0)),
            pl.BlockSpec((D_IN, DH), lambda i: (0, 0)),
            pl.BlockSpec((8, DH), lambda i: (0, 0)),
            pl.BlockSpec((2, RB, DEGW), lambda i: (0, i, 0)),
        ],
        out_specs=[
            pl.BlockSpec((RB, DH), lambda i: (i, 0)),
            pl.BlockSpec((2, RB, DHH), lambda i: (0, i, 0)),
            pl.BlockSpec((RB, DH), lambda i: (i, 0)),
        ],
        out_shape=[
            jax.ShapeDtypeStruct((NPAD, DH), jnp.float32),
            jax.ShapeDtypeStruct((NC, NPAD, DHH), jnp.float32),
            jax.ShapeDtypeStruct((NPAD, DH), jnp.float32),
        ],
    )


def _upd_body(beta, last, sb, x0b, Db, wcb, w1b, b1b, ob):
    S = jnp.concatenate([sb[0], sb[1]], axis=1)
    t = (1.0 - ALPHA) * (Db[...] * S) + ALPHA * x0b[...]
    u = (1.0 - beta) * t + beta * jnp.dot(
        t, wcb[...], preferred_element_type=jnp.float32)
    h = jnp.maximum(u, 0.0)
    if last:
        ob[...] = jnp.dot(h, w1b[...],
                          preferred_element_type=jnp.float32) + b1b[0:1, :]
    else:
        hs = Db[...] * h
        ob[...] = jnp.stack([hs[:, :DHH], hs[:, DHH:]], axis=0)


@functools.cache
def _upd_kernel(beta, last):
    if last:
        out_spec = pl.BlockSpec((RB, DH), lambda i: (i, 0))
        out_shape = jax.ShapeDtypeStruct((NPAD, DH), jnp.float32)
    else:
        out_spec = pl.BlockSpec((2, RB, DHH), lambda i: (0, i, 0))
        out_shape = jax.ShapeDtypeStruct((NC, NPAD, DHH), jnp.float32)
    return pl.pallas_call(
        functools.partial(_upd_body, beta, last),
        grid=(NPAD // RB,),
        in_specs=[
            pl.BlockSpec((2, RB, DHH), lambda i: (0, i, 0)),
            pl.BlockSpec((RB, DH), lambda i: (i, 0)),
            pl.BlockSpec((RB, DH), lambda i: (i, 0)),
            pl.BlockSpec((DH, DH), lambda i: (0, 0)),
            pl.BlockSpec((DH, DH), lambda i: (0, 0)),
            pl.BlockSpec((8, DH), lambda i: (0, 0)),
        ],
        out_specs=out_spec,
        out_shape=out_shape,
    )


def kernel(x, edge_index, W0, b0, W1, b1, Wc):
    deg_kernel, prop_kernel = _sc_kernels()

    pad_blk = jnp.stack([
        jnp.zeros((EPAD - E,), jnp.int32),
        jnp.full((EPAD - E,), NPAD - 1, jnp.int32)])
    eiR = jnp.concatenate(
        [edge_index.astype(jnp.int32), pad_blk], axis=1).reshape(2, EPAD // B, B)
    ones2 = jnp.ones((RPT, DEGW), jnp.float32)
    b0_8 = jnp.tile(b0[None, :], (8, 1))
    b1_8 = jnp.tile(b1[None, :], (8, 1))

    deg = deg_kernel(eiR, ones2)
    x0, hs2, D = _pre_kernel()(x, W0, b0_8, deg)
    for l in range(L):
        beta = float(np.log(THETA / (l + 1) + 1.0))
        s2 = prop_kernel(hs2, eiR)
        hs2 = _upd_kernel(beta, l == L - 1)(s2, x0, D, Wc[l], W1, b1_8)
    return hs2[:N]
